# baseline (speedup 1.0000x reference)
import jax
import jax.numpy as jnp
from jax.experimental import pallas as pl
from jax.experimental.pallas import tpu as pltpu

A = 9
C = 512
H = 50
W = 75
HP = 52
WP = 80
P = HP * WP
B0 = 88
PB = P + 2 * B0
NCHUNK = 4
PC = P // NCHUNK
HC = PC // WP


def _rpn_kernel(xt_ref, w2_ref, bsw_ref, wr_ref, wc_ref, br_ref, bc_ref,
                oT36_ref, oT18_ref, xb_s, xp_s, xn_s, fv_s):
    xb_s[...] = jnp.zeros((PB, C), jnp.bfloat16)
    for h in range(H):
        xb_s[pl.ds(B0 + (h + 1) * WP + 1, W), :] = \
            xt_ref[h].astype(jnp.bfloat16)
    xp_s[pl.ds(1, PB - 1), :] = xb_s[0:PB - 1, :]
    xn_s[pl.ds(0, PB - 1), :] = xb_s[1:PB, :]
    wcat = jnp.concatenate([w2_ref[k].astype(jnp.bfloat16)
                            for k in range(9)], axis=1)
    dn_t = (((1,), (1,)), ((), ()))
    for c in range(NCHUNK):
        base = c * PC
        xcat = jnp.concatenate(
            [(xp_s, xb_s, xn_s)[kw][pl.ds(B0 + base + (kh - 1) * WP, PC), :]
             for kh in range(3) for kw in range(3)], axis=1)
        acc = jax.lax.dot_general(xcat, wcat, dn_t,
                                  preferred_element_type=jnp.float32)
        feat = jnp.maximum(acc + bsw_ref[0, :][None, :],
                           0.0).astype(jnp.bfloat16)
        for hh in range(HC):
            hpad = c * HC + hh
            if 1 <= hpad <= H:
                fv_s[pl.ds((hpad - 1) * W, W), :] = \
                    feat[hh * WP + 1:hh * WP + 1 + W, :]
    whb = jnp.concatenate(
        [jnp.reshape(wr_ref[...], (36, C)),
         jnp.reshape(wc_ref[...], (18, C))], axis=0).astype(jnp.bfloat16)
    oT = jax.lax.dot_general(whb, fv_s[...], dn_t,
                             preferred_element_type=jnp.float32)
    bcat = jnp.concatenate([br_ref[...], bc_ref[...]], axis=1)
    oT = oT + jnp.transpose(bcat, (1, 0))
    oT36_ref[...] = oT[0:36, :]
    oT18_ref[...] = oT[36:54, :]


def kernel(x, W_sw, b_sw, W_cls, b_cls, W_reg, b_reg):
    xt = jnp.transpose(x[0], (1, 2, 0))
    w2 = jnp.transpose(W_sw, (2, 3, 0, 1)).reshape(9, C, C)

    oT36, oT18 = pl.pallas_call(
        _rpn_kernel,
        out_shape=(jax.ShapeDtypeStruct((36, H * W), jnp.float32),
                   jax.ShapeDtypeStruct((18, H * W), jnp.float32)),
        scratch_shapes=[pltpu.VMEM((PB, C), jnp.bfloat16),
                        pltpu.VMEM((PB, C), jnp.bfloat16),
                        pltpu.VMEM((PB, C), jnp.bfloat16),
                        pltpu.VMEM((H * W, C), jnp.bfloat16)],
    )(xt, w2, b_sw.reshape(1, C),
      W_reg.reshape(36 * 4, C // 4), W_cls.reshape(18 * 4, C // 4),
      b_reg.reshape(1, 36), b_cls.reshape(1, 18))

    reg = jnp.transpose(oT36.reshape(A, 4, H * W),
                        (2, 0, 1)).reshape(1, H * W * A, 4)
    cls = jnp.transpose(oT18.reshape(A, 2, H * W),
                        (2, 0, 1)).reshape(1, H * W * A, 2)
    return (reg, cls)


# speedup vs baseline: 1.4595x; 1.4595x over previous
import jax
import jax.numpy as jnp
from jax.experimental import pallas as pl
from jax.experimental.pallas import tpu as pltpu

A = 9
C = 512
H = 50
W = 75
HP = 52
WP = 80
P = HP * WP
B0 = 88
PB = P + 2 * B0
NCHUNK = 4
PC = P // NCHUNK
HC = PC // WP


def _rpn_kernel(xt_ref, w2_ref, bsw_ref, wr_ref, wc_ref, br_ref, bc_ref,
                oT36_ref, oT18_ref, xb_s, xp_s, xn_s, fv_s):
    xb_s[...] = jnp.zeros((PB, C), jnp.bfloat16)
    for h in range(H):
        xb_s[pl.ds(B0 + (h + 1) * WP + 1, W), :] = xt_ref[h]
    xp_s[pl.ds(1, PB - 1), :] = xb_s[0:PB - 1, :]
    xn_s[pl.ds(0, PB - 1), :] = xb_s[1:PB, :]
    wcat = jnp.concatenate([w2_ref[k].astype(jnp.bfloat16)
                            for k in range(9)], axis=1)
    dn_t = (((1,), (1,)), ((), ()))
    for c in range(NCHUNK):
        base = c * PC
        xcat = jnp.concatenate(
            [(xp_s, xb_s, xn_s)[kw][pl.ds(B0 + base + (kh - 1) * WP, PC), :]
             for kh in range(3) for kw in range(3)], axis=1)
        acc = jax.lax.dot_general(xcat, wcat, dn_t,
                                  preferred_element_type=jnp.float32)
        feat = jnp.maximum(acc + bsw_ref[0, :][None, :],
                           0.0).astype(jnp.bfloat16)
        for hh in range(HC):
            hpad = c * HC + hh
            if 1 <= hpad <= H:
                fv_s[pl.ds((hpad - 1) * W, W), :] = \
                    feat[hh * WP + 1:hh * WP + 1 + W, :]
    whb = jnp.concatenate(
        [jnp.reshape(wr_ref[...], (36, C)),
         jnp.reshape(wc_ref[...], (18, C))], axis=0).astype(jnp.bfloat16)
    oT = jax.lax.dot_general(whb, fv_s[...], dn_t,
                             preferred_element_type=jnp.float32)
    bcat = jnp.concatenate([br_ref[...], bc_ref[...]], axis=1)
    oT = oT + jnp.transpose(bcat, (1, 0))
    oT36_ref[...] = oT[0:36, :]
    oT18_ref[...] = oT[36:54, :]


def kernel(x, W_sw, b_sw, W_cls, b_cls, W_reg, b_reg):
    xt = jnp.transpose(x[0], (1, 2, 0)).astype(jnp.bfloat16)
    w2 = jnp.transpose(W_sw, (2, 3, 0, 1)).reshape(9, C, C)

    oT36, oT18 = pl.pallas_call(
        _rpn_kernel,
        out_shape=(jax.ShapeDtypeStruct((36, H * W), jnp.float32),
                   jax.ShapeDtypeStruct((18, H * W), jnp.float32)),
        scratch_shapes=[pltpu.VMEM((PB, C), jnp.bfloat16),
                        pltpu.VMEM((PB, C), jnp.bfloat16),
                        pltpu.VMEM((PB, C), jnp.bfloat16),
                        pltpu.VMEM((H * W, C), jnp.bfloat16)],
    )(xt, w2, b_sw.reshape(1, C),
      W_reg.reshape(36 * 4, C // 4), W_cls.reshape(18 * 4, C // 4),
      b_reg.reshape(1, 36), b_cls.reshape(1, 18))

    reg = jnp.transpose(oT36.reshape(A, 4, H * W),
                        (2, 0, 1)).reshape(1, H * W * A, 4)
    cls = jnp.transpose(oT18.reshape(A, 2, H * W),
                        (2, 0, 1)).reshape(1, H * W * A, 2)
    return (reg, cls)
